# Initial kernel scaffold; baseline (speedup 1.0000x reference)
import jax
import jax.numpy as jnp
from jax.experimental import pallas as pl


def kernel(x, edge_index, W0a, b0a, W0b, b0b, W1a, b1a, W1b, b1b):
    raise NotImplementedError("write your pallas kernel here")


# speedup vs baseline: 4.2217x; 4.2217x over previous
import functools

import jax
import jax.numpy as jnp
from jax import lax
from jax.experimental import pallas as pl
from jax.experimental.pallas import tpu as pltpu
from jax.experimental.pallas import tpu_sc as plsc

N = 10000
E = 320000
D = 128

NC = 2
NS = 16
NW = NC * NS
CH = 128
EPT = 10112
E_PAD = NW * EPT
NCHUNK = EPT // CH
NPAD = 10240
RPT = NPAD // NS
DUMMY_DST = NPAD - 8


def _seg_body(x_hbm, src_hbm, dst_hbm, zeros_hbm, out_hbm,
              src_v, dst_v, rows_v, zero_v, agg_sh, gsem):
    c = lax.axis_index("c")
    s = lax.axis_index("s")
    tile = c * NS + s
    base = tile * EPT

    pltpu.sync_copy(zeros_hbm, zero_v)
    row0 = s * RPT
    for k in range(RPT // CH):
        pltpu.sync_copy(zero_v, agg_sh.at[pl.ds(row0 + k * CH, CH)])
    plsc.subcore_barrier()

    def body(j, carry):
        off = pl.multiple_of(base + j * CH, 8)
        pltpu.sync_copy(src_hbm.at[pl.ds(off, CH)], src_v)
        pltpu.sync_copy(dst_hbm.at[pl.ds(off, CH)], dst_v)
        pltpu.async_copy(x_hbm.at[src_v], rows_v, gsem).wait()
        pltpu.sync_copy(rows_v, agg_sh.at[dst_v], add=True)
        return carry

    lax.fori_loop(0, NCHUNK, body, 0)
    plsc.subcore_barrier()

    pltpu.sync_copy(agg_sh.at[pl.ds(row0, RPT)],
                    out_hbm.at[c, pl.ds(row0, RPT)])


_segsum = functools.partial(
    pl.kernel,
    mesh=plsc.VectorSubcoreMesh(core_axis_name="c", subcore_axis_name="s"),
    out_type=jax.ShapeDtypeStruct((NC, NPAD, D), jnp.float32),
    scratch_types=[
        pltpu.VMEM((CH,), jnp.int32),
        pltpu.VMEM((CH,), jnp.int32),
        pltpu.VMEM((CH, D), jnp.float32),
        pltpu.VMEM((CH, D), jnp.float32),
        pltpu.VMEM_SHARED((NPAD, D), jnp.float32),
        pltpu.SemaphoreType.DMA,
    ],
)(_seg_body)


BM = 1000


def _mlp_body(x_ref, p_ref, wa_ref, ba_ref, wb_ref, bb_ref, o_ref):
    t = x_ref[...] + p_ref[0] + p_ref[1]
    u = jnp.maximum(
        jnp.dot(t, wa_ref[...], preferred_element_type=jnp.float32)
        + ba_ref[...], 0.0)
    v = jnp.dot(u, wb_ref[...], preferred_element_type=jnp.float32) \
        + bb_ref[...]
    o_ref[...] = jnp.maximum(v, 0.0)


def _gin_dense(x, p, wa, ba, wb, bb):
    return pl.pallas_call(
        _mlp_body,
        grid=(N // BM,),
        in_specs=[
            pl.BlockSpec((BM, D), lambda i: (i, 0)),
            pl.BlockSpec((2, BM, D), lambda i: (0, i, 0)),
            pl.BlockSpec((D, D), lambda i: (0, 0)),
            pl.BlockSpec((1, D), lambda i: (0, 0)),
            pl.BlockSpec((D, D), lambda i: (0, 0)),
            pl.BlockSpec((1, D), lambda i: (0, 0)),
        ],
        out_specs=pl.BlockSpec((BM, D), lambda i: (i, 0)),
        out_shape=jax.ShapeDtypeStruct((N, D), jnp.float32),
    )(x, p, wa, ba, wb, bb)


@jax.jit
def kernel(x, edge_index, W0a, b0a, W0b, b0b, W1a, b1a, W1b, b1b):
    pad = E_PAD - E
    src = jnp.concatenate([edge_index[0],
                           jnp.zeros((pad,), jnp.int32)])
    dst = jnp.concatenate([edge_index[1],
                           jnp.full((pad,), DUMMY_DST, jnp.int32)])
    zeros = jnp.zeros((CH, D), jnp.float32)

    p = _segsum(x, src, dst, zeros)
    h = _gin_dense(x, p, W0a, b0a.reshape(1, D), W0b, b0b.reshape(1, D))
    p2 = _segsum(h, src, dst, zeros)
    out = _gin_dense(h, p2, W1a, b1a.reshape(1, D), W1b, b1b.reshape(1, D))
    return out
